# Initial kernel scaffold; baseline (speedup 1.0000x reference)
import jax
import jax.numpy as jnp
from jax.experimental import pallas as pl


def kernel(x, edge_index, edge_attr, face_index, face_attr, g, node_batch, We1, be1, lne_g, lne_b, We2, be2, Wn1, bn1, lnn_g, lnn_b, Wn2, bn2, Wg1, bg1, lng_g, lng_b, Wg2, bg2, norm_g, norm_b):
    raise NotImplementedError("write your pallas kernel here")


# speedup vs baseline: 3.2900x; 3.2900x over previous
import functools

import jax
import jax.numpy as jnp
from jax import lax
from jax.experimental import pallas as pl
from jax.experimental.pallas import tpu as pltpu
from jax.experimental.pallas import tpu_sc as plsc

F32 = jnp.float32
I32 = jnp.int32

_N = 10000
_NPAD = 10240
_E = 320000
_B = 64
_DN = 128
_DE = 16
_DG = 128

_NW = 32
_CHUNK = 80
_CHUNKS_PER_W = _E // (_NW * _CHUNK)
_RPT = _NPAD // 16


def _layernorm(a, gamma, beta):
    mu = jnp.mean(a, axis=-1, keepdims=True)
    var = jnp.mean((a - mu) ** 2, axis=-1, keepdims=True)
    return (a - mu) * lax.rsqrt(var + 1e-5) * gamma + beta


def _sc_gather_scatter_body(x_hbm, gi_hbm, si_hbm, z_hbm,
                            gath_out, acc_out,
                            idx_g, idx_s, rows_v, sem, acc_sh):
    c = lax.axis_index("c")
    s = lax.axis_index("s")
    wid = s * 2 + c
    pltpu.sync_copy(z_hbm.at[pl.ds(s * _RPT, _RPT)],
                    acc_sh.at[pl.ds(s * _RPT, _RPT)])
    plsc.subcore_barrier()

    def step(i, carry):
        r = wid * _CHUNKS_PER_W + i
        pltpu.sync_copy(gi_hbm.at[r], idx_g)
        pltpu.async_copy(x_hbm.at[idx_g], rows_v, sem).wait()
        pltpu.sync_copy(rows_v, gath_out.at[pl.ds(r * _CHUNK, _CHUNK)])
        pltpu.sync_copy(si_hbm.at[r], idx_s)
        pltpu.sync_copy(rows_v, acc_sh.at[idx_s], add=True)
        return carry

    lax.fori_loop(0, _CHUNKS_PER_W, step, 0)
    plsc.subcore_barrier()
    pltpu.sync_copy(acc_sh.at[pl.ds(s * _RPT, _RPT)],
                    acc_out.at[c, pl.ds(s * _RPT, _RPT)])


def _sc_gather_scatter(x, gi2, si2, zeros_big):
    mesh = plsc.VectorSubcoreMesh(core_axis_name="c", subcore_axis_name="s")
    k = functools.partial(
        pl.kernel,
        mesh=mesh,
        out_type=[jax.ShapeDtypeStruct((_E, _DN), F32),
                  jax.ShapeDtypeStruct((2, _NPAD, _DN), F32)],
        scratch_types=[pltpu.VMEM((_CHUNK,), I32),
                       pltpu.VMEM((_CHUNK,), I32),
                       pltpu.VMEM((_CHUNK, _DN), F32),
                       pltpu.SemaphoreType.DMA,
                       pltpu.VMEM_SHARED((_NPAD, _DN), F32)],
    )(_sc_gather_scatter_body)
    return k(x, gi2, si2, zeros_big)


def _sc_scatter_e_body(eb_hbm, row_hbm, col_hbm, z_hbm,
                       acc_out,
                       ev, bufa, bufb, idx_r, idx_c, acc_sh):
    c = lax.axis_index("c")
    s = lax.axis_index("s")
    wid = s * 2 + c
    pltpu.sync_copy(z_hbm.at[pl.ds(s * _RPT, _RPT)],
                    acc_sh.at[pl.ds(s * _RPT, _RPT)])
    pltpu.sync_copy(z_hbm.at[pl.ds(0, _CHUNK)], bufa)
    pltpu.sync_copy(z_hbm.at[pl.ds(0, _CHUNK)], bufb)
    plsc.subcore_barrier()

    def step(i, carry):
        r = wid * _CHUNKS_PER_W + i
        pltpu.sync_copy(eb_hbm.at[pl.ds(r * _CHUNK, _CHUNK)], ev)

        def stage(j, cc):
            v = ev[j, pl.ds(0, _DE)]
            bufa[j, pl.ds(0, _DE)] = v
            bufb[j, pl.ds(_DE, _DE)] = v
            return cc

        lax.fori_loop(0, _CHUNK, stage, 0)
        pltpu.sync_copy(col_hbm.at[r], idx_c)
        pltpu.sync_copy(row_hbm.at[r], idx_r)
        pltpu.sync_copy(bufa, acc_sh.at[idx_c], add=True)
        pltpu.sync_copy(bufb, acc_sh.at[idx_r], add=True)
        return carry

    lax.fori_loop(0, _CHUNKS_PER_W, step, 0)
    plsc.subcore_barrier()
    pltpu.sync_copy(acc_sh.at[pl.ds(s * _RPT, _RPT)],
                    acc_out.at[c, pl.ds(s * _RPT, _RPT)])


def _sc_scatter_e(ebar, row2, col2, zeros_big):
    mesh = plsc.VectorSubcoreMesh(core_axis_name="c", subcore_axis_name="s")
    k = functools.partial(
        pl.kernel,
        mesh=mesh,
        out_type=[jax.ShapeDtypeStruct((2, _NPAD, _DN), F32)],
        scratch_types=[pltpu.VMEM((_CHUNK, _DE), F32),
                       pltpu.VMEM((_CHUNK, _DN), F32),
                       pltpu.VMEM((_CHUNK, _DN), F32),
                       pltpu.VMEM((_CHUNK,), I32),
                       pltpu.VMEM((_CHUNK,), I32),
                       pltpu.VMEM_SHARED((_NPAD, _DN), F32)],
    )(_sc_scatter_e_body)
    return k(ebar, row2, col2, zeros_big)


def _pre_body(nb2, g, W1ge, W1gn, sh_o, gwe_o, gwn_o):
    nbv = nb2[...]

    def st(k, carry):
        cval = jnp.sum((nbv < (k + 1)).astype(I32))
        sh_o[pl.ds(k, 1), :] = jnp.full((1, 1), cval, I32)
        return carry

    lax.fori_loop(0, 64, st, 0)
    gwe_o[...] = jnp.dot(g[...], W1ge[...], preferred_element_type=F32)
    gwn_o[...] = jnp.dot(g[...], W1gn[...], preferred_element_type=F32)


def _tc_pre(nb2, g, W1ge, W1gn):
    return pl.pallas_call(
        _pre_body,
        out_shape=[jax.ShapeDtypeStruct((64, 1), I32),
                   jax.ShapeDtypeStruct((_B, 4 * _DE), F32),
                   jax.ShapeDtypeStruct((_B, 4 * _DN), F32)],
    )(nb2, g, W1ge, W1gn)


_BE = 512


def _edge_body(xr, xc, ea, row3, sh, gwe, Wr, Wc, Wa, be1, lg, lb, W2, b2,
               ebar_o, enew_o):
    row = row3[0]
    m = (row >= sh[...]).astype(I32)
    ebi = jnp.sum(m, axis=0, keepdims=True)
    oh = (lax.broadcasted_iota(I32, (64, _BE), 0) == ebi).astype(F32)
    ge = lax.dot_general(oh, gwe[...], (((0,), (0,)), ((), ())),
                         preferred_element_type=F32)
    h = (jnp.dot(xr[...], Wr[...], preferred_element_type=F32)
         + jnp.dot(xc[...], Wc[...], preferred_element_type=F32)
         + jnp.dot(ea[...], Wa[...], preferred_element_type=F32)
         + ge + be1[...])
    a = jax.nn.gelu(h)
    ln = _layernorm(a, lg[...], lb[...])
    e = jnp.dot(ln, W2[...], preferred_element_type=F32) + b2[...]
    ebar_o[...] = e
    enew_o[...] = ea[...] + e


def _tc_edge(xr, xc, ea, row3, sh, gwe, Wr, Wc, Wa, be1, lg, lb, W2, b2):
    nblk = _E // _BE
    full = lambda i: (0, 0)
    return pl.pallas_call(
        _edge_body,
        grid=(nblk,),
        in_specs=[
            pl.BlockSpec((_BE, _DN), lambda i: (i, 0)),
            pl.BlockSpec((_BE, _DN), lambda i: (i, 0)),
            pl.BlockSpec((_BE, _DE), lambda i: (i, 0)),
            pl.BlockSpec((1, 1, _BE), lambda i: (i, 0, 0)),
            pl.BlockSpec((64, 1), full),
            pl.BlockSpec((_B, 4 * _DE), full),
            pl.BlockSpec((_DN, 4 * _DE), full),
            pl.BlockSpec((_DN, 4 * _DE), full),
            pl.BlockSpec((_DE, 4 * _DE), full),
            pl.BlockSpec((1, 4 * _DE), full),
            pl.BlockSpec((1, 4 * _DE), full),
            pl.BlockSpec((1, 4 * _DE), full),
            pl.BlockSpec((4 * _DE, _DE), full),
            pl.BlockSpec((1, _DE), full),
        ],
        out_specs=[pl.BlockSpec((_BE, _DE), lambda i: (i, 0)),
                   pl.BlockSpec((_BE, _DE), lambda i: (i, 0))],
        out_shape=[jax.ShapeDtypeStruct((_E, _DE), F32),
                   jax.ShapeDtypeStruct((_E, _DE), F32)],
        compiler_params=pltpu.CompilerParams(
            dimension_semantics=("arbitrary",)),
    )(xr, xc, ea, row3, sh, gwe, Wr, Wc, Wa, be1, lg, lb, W2, b2)


_BN = 512


def _node_body(xb, aix, aox, ae, nb3, gwn,
               Wx, Wie, Wix, Woe, Wox, bn1, lg, lb, Wn2, bn2, ng, nbb,
               xnew_o, sumx_o, sume_o):
    i = pl.program_id(0)
    ix = aix[0] + aix[1]
    ox = aox[0] + aox[1]
    acce = ae[0] + ae[1]
    ie = acce[:, 0:_DE]
    oe = acce[:, _DE:2 * _DE]
    nbv = nb3[0]
    oh = (lax.broadcasted_iota(I32, (64, _BN), 0) == nbv).astype(F32)
    gterm = lax.dot_general(oh, gwn[...], (((0,), (0,)), ((), ())),
                            preferred_element_type=F32)
    pre = (jnp.dot(xb[...], Wx[...], preferred_element_type=F32)
           + jnp.dot(ie, Wie[...], preferred_element_type=F32)
           + jnp.dot(ix, Wix[...], preferred_element_type=F32)
           + jnp.dot(oe, Woe[...], preferred_element_type=F32)
           + jnp.dot(ox, Wox[...], preferred_element_type=F32)
           + gterm + bn1[...])
    a = jax.nn.gelu(pre)
    ln = _layernorm(a, lg[...], lb[...])
    x_bar = jnp.dot(ln, Wn2[...], preferred_element_type=F32) + bn2[...]
    xnew_o[...] = _layernorm(x_bar, ng[...], nbb[...]) + xb[...]

    @pl.when(i == 0)
    def _init():
        sumx_o[...] = jnp.zeros_like(sumx_o)
        sume_o[...] = jnp.zeros_like(sume_o)

    sumx_o[...] += lax.dot_general(oh, x_bar, (((1,), (0,)), ((), ())),
                                   preferred_element_type=F32)
    sume_o[...] += lax.dot_general(oh, oe, (((1,), (0,)), ((), ())),
                                   preferred_element_type=F32)


def _tc_node(xp, aix, aox, ae, nb3, gwn,
             Wx, Wie, Wix, Woe, Wox, bn1, lg, lb, Wn2, bn2, ng, nbb):
    nblk = _NPAD // _BN
    full = lambda i: (0, 0)
    h = 4 * _DN
    return pl.pallas_call(
        _node_body,
        grid=(nblk,),
        in_specs=[
            pl.BlockSpec((_BN, _DN), lambda i: (i, 0)),
            pl.BlockSpec((2, _BN, _DN), lambda i: (0, i, 0)),
            pl.BlockSpec((2, _BN, _DN), lambda i: (0, i, 0)),
            pl.BlockSpec((2, _BN, _DN), lambda i: (0, i, 0)),
            pl.BlockSpec((1, 1, _BN), lambda i: (i, 0, 0)),
            pl.BlockSpec((_B, h), full),
            pl.BlockSpec((_DN, h), full),
            pl.BlockSpec((_DE, h), full),
            pl.BlockSpec((_DN, h), full),
            pl.BlockSpec((_DE, h), full),
            pl.BlockSpec((_DN, h), full),
            pl.BlockSpec((1, h), full),
            pl.BlockSpec((1, h), full),
            pl.BlockSpec((1, h), full),
            pl.BlockSpec((h, _DN), full),
            pl.BlockSpec((1, _DN), full),
            pl.BlockSpec((1, _DN), full),
            pl.BlockSpec((1, _DN), full),
        ],
        out_specs=[pl.BlockSpec((_BN, _DN), lambda i: (i, 0)),
                   pl.BlockSpec((_B, _DN), full),
                   pl.BlockSpec((_B, _DE), full)],
        out_shape=[jax.ShapeDtypeStruct((_NPAD, _DN), F32),
                   jax.ShapeDtypeStruct((_B, _DN), F32),
                   jax.ShapeDtypeStruct((_B, _DE), F32)],
        compiler_params=pltpu.CompilerParams(
            dimension_semantics=("arbitrary",)),
    )(xp, aix, aox, ae, nb3, gwn,
      Wx, Wie, Wix, Woe, Wox, bn1, lg, lb, Wn2, bn2, ng, nbb)


def _global_body(g, sumx, sume, Wgg, Wgx, Wge, bg1, lg, lb, Wg2, bg2, gnew_o):
    pre = (jnp.dot(g[...], Wgg[...], preferred_element_type=F32)
           + jnp.dot(sumx[...], Wgx[...], preferred_element_type=F32)
           + jnp.dot(sume[...], Wge[...], preferred_element_type=F32)
           + bg1[...])
    a = jax.nn.gelu(pre)
    ln = _layernorm(a, lg[...], lb[...])
    gnew_o[...] = g[...] + jnp.dot(ln, Wg2[...],
                                   preferred_element_type=F32) + bg2[...]


def _tc_global(g, sumx, sume, Wgg, Wgx, Wge, bg1, lg, lb, Wg2, bg2):
    return pl.pallas_call(
        _global_body,
        out_shape=jax.ShapeDtypeStruct((_B, _DG), F32),
    )(g, sumx, sume, Wgg, Wgx, Wge, bg1, lg, lb, Wg2, bg2)


def kernel(x, edge_index, edge_attr, face_index, face_attr, g, node_batch,
           We1, be1, lne_g, lne_b, We2, be2,
           Wn1, bn1, lnn_g, lnn_b, Wn2, bn2,
           Wg1, bg1, lng_g, lng_b, Wg2, bg2,
           norm_g, norm_b):
    row = edge_index[0]
    col = edge_index[1]
    row2 = row.reshape(_E // _CHUNK, _CHUNK)
    col2 = col.reshape(_E // _CHUNK, _CHUNK)
    row3 = row.reshape(_E // _BE, 1, _BE)

    nb_pad = jnp.pad(node_batch, (0, _NPAD - _N), constant_values=_B)
    nb2 = nb_pad.reshape(_NPAD // 128, 128)
    nb3 = nb_pad.reshape(_NPAD // _BN, 1, _BN)
    x_pad = jnp.pad(x, ((0, _NPAD - _N), (0, 0)))

    zeros_big = jnp.zeros((_NPAD, _DN), F32)

    Wr, Wc, Wa, W1ge = (We1[:_DN], We1[_DN:2 * _DN],
                        We1[2 * _DN:2 * _DN + _DE], We1[2 * _DN + _DE:])
    o = 0
    Wx = Wn1[o:o + _DN]; o += _DN
    Wie = Wn1[o:o + _DE]; o += _DE
    Wix = Wn1[o:o + _DN]; o += _DN
    Woe = Wn1[o:o + _DE]; o += _DE
    Wox = Wn1[o:o + _DN]; o += _DN
    W1gn = Wn1[o:]
    Wgg, Wgx, Wge = Wg1[:_DG], Wg1[_DG:_DG + _DN], Wg1[_DG + _DN:]

    r1 = lambda v: v.reshape(1, -1)

    sh, gwe, gwn = _tc_pre(nb2, g, W1ge, W1gn)

    xr, acc_in_x = _sc_gather_scatter(x, row2, col2, zeros_big)
    xc, acc_out_x = _sc_gather_scatter(x, col2, row2, zeros_big)

    e_bar, edge_new = _tc_edge(xr, xc, edge_attr, row3, sh, gwe,
                               Wr, Wc, Wa, r1(be1), r1(lne_g), r1(lne_b),
                               We2, r1(be2))

    acc_e, = _sc_scatter_e(e_bar, row2, col2, zeros_big)

    x_new_pad, sum_x, sum_e = _tc_node(
        x_pad, acc_in_x, acc_out_x, acc_e, nb3, gwn,
        Wx, Wie, Wix, Woe, Wox, r1(bn1), r1(lnn_g), r1(lnn_b),
        Wn2, r1(bn2), r1(norm_g), r1(norm_b))

    g_new = _tc_global(g, sum_x, sum_e, Wgg, Wgx, Wge,
                       r1(bg1), r1(lng_g), r1(lng_b), Wg2, r1(bg2))

    return (x_new_pad[:_N], edge_new, face_attr, g_new)


# speedup vs baseline: 4.5005x; 1.3679x over previous
import functools

import jax
import jax.numpy as jnp
from jax import lax
from jax.experimental import pallas as pl
from jax.experimental.pallas import tpu as pltpu
from jax.experimental.pallas import tpu_sc as plsc

F32 = jnp.float32
I32 = jnp.int32

_N = 10000
_NPAD = 10240
_E = 320000
_B = 64
_DN = 128
_DE = 16
_DG = 128

_NW = 32
_CHUNK = 80
_CHUNKS_PER_W = _E // (_NW * _CHUNK)
_RPT = _NPAD // 16


def _layernorm(a, gamma, beta):
    mu = jnp.mean(a, axis=-1, keepdims=True)
    var = jnp.mean((a - mu) ** 2, axis=-1, keepdims=True)
    return (a - mu) * lax.rsqrt(var + 1e-5) * gamma + beta


_U = 4
_UE = 2
_CHUNK_E = 40
_CPW_E = _E // (_NW * _CHUNK_E)


def _sc_gather_scatter_body(x_hbm, gi_hbm, si_hbm, z_hbm,
                            gath_out, acc_out,
                            idx_g, idx_s, rows_v, acc_sh,
                            gs0, gs1, gs2, gs3,
                            ws0, ws1, ws2, ws3):
    gsems = (gs0, gs1, gs2, gs3)
    wsems = (ws0, ws1, ws2, ws3)
    c = lax.axis_index("c")
    s = lax.axis_index("s")
    wid = s * 2 + c
    pltpu.sync_copy(z_hbm.at[pl.ds(s * _RPT, _RPT)],
                    acc_sh.at[pl.ds(s * _RPT, _RPT)])
    plsc.subcore_barrier()

    def group(base, n):
        gh = []
        for b in range(n):
            r = base + b
            pltpu.sync_copy(gi_hbm.at[pl.ds(r * _CHUNK, _CHUNK)],
                            idx_g.at[b])
            gh.append(pltpu.async_copy(x_hbm.at[idx_g.at[b]],
                                       rows_v.at[b], gsems[b]))
        wh = []
        for b in range(n):
            r = base + b
            gh[b].wait()
            wh.append(pltpu.async_copy(rows_v.at[b],
                                       gath_out.at[pl.ds(r * _CHUNK, _CHUNK)],
                                       wsems[b]))
            pltpu.sync_copy(si_hbm.at[pl.ds(r * _CHUNK, _CHUNK)], idx_s)
            pltpu.sync_copy(rows_v.at[b], acc_sh.at[idx_s], add=True)
        for b in range(n):
            wh[b].wait()

    def step(k, carry):
        group(wid * _CHUNKS_PER_W + k * _U, _U)
        return carry

    ngrp = _CHUNKS_PER_W // _U
    lax.fori_loop(0, ngrp, step, 0)
    if _CHUNKS_PER_W % _U:
        group(wid * _CHUNKS_PER_W + ngrp * _U, _CHUNKS_PER_W % _U)
    plsc.subcore_barrier()
    pltpu.sync_copy(acc_sh.at[pl.ds(s * _RPT, _RPT)],
                    acc_out.at[c, pl.ds(s * _RPT, _RPT)])


def _sc_gather_scatter(x, gi2, si2, zeros_big):
    mesh = plsc.VectorSubcoreMesh(core_axis_name="c", subcore_axis_name="s")
    k = functools.partial(
        pl.kernel,
        mesh=mesh,
        out_type=[jax.ShapeDtypeStruct((_E, _DN), F32),
                  jax.ShapeDtypeStruct((2, _NPAD, _DN), F32)],
        scratch_types=[pltpu.VMEM((_U, _CHUNK), I32),
                       pltpu.VMEM((_CHUNK,), I32),
                       pltpu.VMEM((_U, _CHUNK, _DN), F32),
                       pltpu.VMEM_SHARED((_NPAD, _DN), F32)]
        + [pltpu.SemaphoreType.DMA] * (2 * _U),
    )(_sc_gather_scatter_body)
    return k(x, gi2, si2, zeros_big)


def _sc_scatter_e_body(eb_hbm, row_hbm, col_hbm, z_hbm,
                       acc_out,
                       ev, bufa, bufb, idx_r, idx_c, acc_sh,
                       sa0, sa1, sb0, sb1):
    sas = (sa0, sa1)
    sbs = (sb0, sb1)
    c = lax.axis_index("c")
    s = lax.axis_index("s")
    wid = s * 2 + c
    pltpu.sync_copy(z_hbm.at[pl.ds(s * _RPT, _RPT)],
                    acc_sh.at[pl.ds(s * _RPT, _RPT)])
    for p in range(_UE):
        pltpu.sync_copy(z_hbm.at[pl.ds(0, _CHUNK_E)], bufa.at[p])
        pltpu.sync_copy(z_hbm.at[pl.ds(0, _CHUNK_E)], bufb.at[p])
    plsc.subcore_barrier()

    def group(base, n):
        hs = []
        for p in range(n):
            r = base + p
            pltpu.sync_copy(eb_hbm.at[pl.ds(r * _CHUNK_E, _CHUNK_E)], ev)

            def stage(j, cc, p=p):
                v = ev[j, pl.ds(0, _DE)]
                bufa[p, j, pl.ds(0, _DE)] = v
                bufb[p, j, pl.ds(_DE, _DE)] = v
                return cc

            lax.fori_loop(0, _CHUNK_E, stage, 0)
            pltpu.sync_copy(col_hbm.at[pl.ds(r * _CHUNK_E, _CHUNK_E)], idx_c.at[p])
            pltpu.sync_copy(row_hbm.at[pl.ds(r * _CHUNK_E, _CHUNK_E)], idx_r.at[p])
            hs.append(pltpu.async_copy(bufa.at[p], acc_sh.at[idx_c.at[p]],
                                       sas[p], add=True))
            hs.append(pltpu.async_copy(bufb.at[p], acc_sh.at[idx_r.at[p]],
                                       sbs[p], add=True))
        for h in hs:
            h.wait()

    def step(k, carry):
        group(wid * _CPW_E + k * _UE, _UE)
        return carry

    ngrp = _CPW_E // _UE
    lax.fori_loop(0, ngrp, step, 0)
    if _CPW_E % _UE:
        group(wid * _CPW_E + ngrp * _UE, _CPW_E % _UE)
    plsc.subcore_barrier()
    pltpu.sync_copy(acc_sh.at[pl.ds(s * _RPT, _RPT)],
                    acc_out.at[c, pl.ds(s * _RPT, _RPT)])


def _sc_scatter_e(ebar, row2, col2, zeros_big):
    mesh = plsc.VectorSubcoreMesh(core_axis_name="c", subcore_axis_name="s")
    k = functools.partial(
        pl.kernel,
        mesh=mesh,
        out_type=[jax.ShapeDtypeStruct((2, _NPAD, _DN), F32)],
        scratch_types=[pltpu.VMEM((_CHUNK_E, _DE), F32),
                       pltpu.VMEM((_UE, _CHUNK_E, _DN), F32),
                       pltpu.VMEM((_UE, _CHUNK_E, _DN), F32),
                       pltpu.VMEM((_UE, _CHUNK_E), I32),
                       pltpu.VMEM((_UE, _CHUNK_E), I32),
                       pltpu.VMEM_SHARED((_NPAD, _DN), F32)]
        + [pltpu.SemaphoreType.DMA] * (2 * _UE),
    )(_sc_scatter_e_body)
    return k(ebar, row2, col2, zeros_big)


def _pre_body(nb2, g, W1ge, W1gn, sh_o, gwe_o, gwn_o):
    nbv = nb2[...]

    def st(k, carry):
        cval = jnp.sum((nbv < (k + 1)).astype(I32))
        sh_o[pl.ds(k, 1), :] = jnp.full((1, 1), cval, I32)
        return carry

    lax.fori_loop(0, 64, st, 0)
    gwe_o[...] = jnp.dot(g[...], W1ge[...], preferred_element_type=F32)
    gwn_o[...] = jnp.dot(g[...], W1gn[...], preferred_element_type=F32)


def _tc_pre(nb2, g, W1ge, W1gn):
    return pl.pallas_call(
        _pre_body,
        out_shape=[jax.ShapeDtypeStruct((64, 1), I32),
                   jax.ShapeDtypeStruct((_B, 4 * _DE), F32),
                   jax.ShapeDtypeStruct((_B, 4 * _DN), F32)],
    )(nb2, g, W1ge, W1gn)


_BE = 1280


def _edge_body(xr, xc, ea, row3, sh, gwe, Wr, Wc, Wa, be1, lg, lb, W2, b2,
               ebar_o, enew_o):
    row = row3[0]
    m = (row >= sh[...]).astype(I32)
    ebi = jnp.sum(m, axis=0, keepdims=True)
    oh = (lax.broadcasted_iota(I32, (64, _BE), 0) == ebi).astype(F32)
    ge = lax.dot_general(oh, gwe[...], (((0,), (0,)), ((), ())),
                         preferred_element_type=F32)
    bf = jnp.bfloat16
    h = (jnp.dot(xr[...].astype(bf), Wr[...].astype(bf),
                 preferred_element_type=F32)
         + jnp.dot(xc[...].astype(bf), Wc[...].astype(bf),
                   preferred_element_type=F32)
         + jnp.dot(ea[...], Wa[...], preferred_element_type=F32)
         + ge + be1[...])
    a = jax.nn.gelu(h)
    ln = _layernorm(a, lg[...], lb[...])
    e = jnp.dot(ln, W2[...], preferred_element_type=F32) + b2[...]
    ebar_o[...] = e
    enew_o[...] = ea[...] + e


def _tc_edge(xr, xc, ea, row3, sh, gwe, Wr, Wc, Wa, be1, lg, lb, W2, b2):
    nblk = _E // _BE
    full = lambda i: (0, 0)
    return pl.pallas_call(
        _edge_body,
        grid=(nblk,),
        in_specs=[
            pl.BlockSpec((_BE, _DN), lambda i: (i, 0)),
            pl.BlockSpec((_BE, _DN), lambda i: (i, 0)),
            pl.BlockSpec((_BE, _DE), lambda i: (i, 0)),
            pl.BlockSpec((1, 1, _BE), lambda i: (i, 0, 0)),
            pl.BlockSpec((64, 1), full),
            pl.BlockSpec((_B, 4 * _DE), full),
            pl.BlockSpec((_DN, 4 * _DE), full),
            pl.BlockSpec((_DN, 4 * _DE), full),
            pl.BlockSpec((_DE, 4 * _DE), full),
            pl.BlockSpec((1, 4 * _DE), full),
            pl.BlockSpec((1, 4 * _DE), full),
            pl.BlockSpec((1, 4 * _DE), full),
            pl.BlockSpec((4 * _DE, _DE), full),
            pl.BlockSpec((1, _DE), full),
        ],
        out_specs=[pl.BlockSpec((_BE, _DE), lambda i: (i, 0)),
                   pl.BlockSpec((_BE, _DE), lambda i: (i, 0))],
        out_shape=[jax.ShapeDtypeStruct((_E, _DE), F32),
                   jax.ShapeDtypeStruct((_E, _DE), F32)],
        compiler_params=pltpu.CompilerParams(
            dimension_semantics=("arbitrary",)),
    )(xr, xc, ea, row3, sh, gwe, Wr, Wc, Wa, be1, lg, lb, W2, b2)


_BN = 512


def _node_body(xb, aix, aox, ae, nb3, gwn,
               Wx, Wie, Wix, Woe, Wox, bn1, lg, lb, Wn2, bn2, ng, nbb,
               xnew_o, sumx_o, sume_o):
    i = pl.program_id(0)
    ix = aix[0] + aix[1]
    ox = aox[0] + aox[1]
    acce = ae[0] + ae[1]
    ie = acce[:, 0:_DE]
    oe = acce[:, _DE:2 * _DE]
    nbv = nb3[0]
    oh = (lax.broadcasted_iota(I32, (64, _BN), 0) == nbv).astype(F32)
    gterm = lax.dot_general(oh, gwn[...], (((0,), (0,)), ((), ())),
                            preferred_element_type=F32)
    pre = (jnp.dot(xb[...], Wx[...], preferred_element_type=F32)
           + jnp.dot(ie, Wie[...], preferred_element_type=F32)
           + jnp.dot(ix, Wix[...], preferred_element_type=F32)
           + jnp.dot(oe, Woe[...], preferred_element_type=F32)
           + jnp.dot(ox, Wox[...], preferred_element_type=F32)
           + gterm + bn1[...])
    a = jax.nn.gelu(pre)
    ln = _layernorm(a, lg[...], lb[...])
    x_bar = jnp.dot(ln, Wn2[...], preferred_element_type=F32) + bn2[...]
    xnew_o[...] = _layernorm(x_bar, ng[...], nbb[...]) + xb[...]

    @pl.when(i == 0)
    def _init():
        sumx_o[...] = jnp.zeros_like(sumx_o)
        sume_o[...] = jnp.zeros_like(sume_o)

    sumx_o[...] += lax.dot_general(oh, x_bar, (((1,), (0,)), ((), ())),
                                   preferred_element_type=F32)
    sume_o[...] += lax.dot_general(oh, oe, (((1,), (0,)), ((), ())),
                                   preferred_element_type=F32)


def _tc_node(xp, aix, aox, ae, nb3, gwn,
             Wx, Wie, Wix, Woe, Wox, bn1, lg, lb, Wn2, bn2, ng, nbb):
    nblk = _NPAD // _BN
    full = lambda i: (0, 0)
    h = 4 * _DN
    return pl.pallas_call(
        _node_body,
        grid=(nblk,),
        in_specs=[
            pl.BlockSpec((_BN, _DN), lambda i: (i, 0)),
            pl.BlockSpec((2, _BN, _DN), lambda i: (0, i, 0)),
            pl.BlockSpec((2, _BN, _DN), lambda i: (0, i, 0)),
            pl.BlockSpec((2, _BN, _DN), lambda i: (0, i, 0)),
            pl.BlockSpec((1, 1, _BN), lambda i: (i, 0, 0)),
            pl.BlockSpec((_B, h), full),
            pl.BlockSpec((_DN, h), full),
            pl.BlockSpec((_DE, h), full),
            pl.BlockSpec((_DN, h), full),
            pl.BlockSpec((_DE, h), full),
            pl.BlockSpec((_DN, h), full),
            pl.BlockSpec((1, h), full),
            pl.BlockSpec((1, h), full),
            pl.BlockSpec((1, h), full),
            pl.BlockSpec((h, _DN), full),
            pl.BlockSpec((1, _DN), full),
            pl.BlockSpec((1, _DN), full),
            pl.BlockSpec((1, _DN), full),
        ],
        out_specs=[pl.BlockSpec((_BN, _DN), lambda i: (i, 0)),
                   pl.BlockSpec((_B, _DN), full),
                   pl.BlockSpec((_B, _DE), full)],
        out_shape=[jax.ShapeDtypeStruct((_NPAD, _DN), F32),
                   jax.ShapeDtypeStruct((_B, _DN), F32),
                   jax.ShapeDtypeStruct((_B, _DE), F32)],
        compiler_params=pltpu.CompilerParams(
            dimension_semantics=("arbitrary",)),
    )(xp, aix, aox, ae, nb3, gwn,
      Wx, Wie, Wix, Woe, Wox, bn1, lg, lb, Wn2, bn2, ng, nbb)


def _global_body(g, sumx, sume, Wgg, Wgx, Wge, bg1, lg, lb, Wg2, bg2, gnew_o):
    pre = (jnp.dot(g[...], Wgg[...], preferred_element_type=F32)
           + jnp.dot(sumx[...], Wgx[...], preferred_element_type=F32)
           + jnp.dot(sume[...], Wge[...], preferred_element_type=F32)
           + bg1[...])
    a = jax.nn.gelu(pre)
    ln = _layernorm(a, lg[...], lb[...])
    gnew_o[...] = g[...] + jnp.dot(ln, Wg2[...],
                                   preferred_element_type=F32) + bg2[...]


def _tc_global(g, sumx, sume, Wgg, Wgx, Wge, bg1, lg, lb, Wg2, bg2):
    return pl.pallas_call(
        _global_body,
        out_shape=jax.ShapeDtypeStruct((_B, _DG), F32),
    )(g, sumx, sume, Wgg, Wgx, Wge, bg1, lg, lb, Wg2, bg2)


def kernel(x, edge_index, edge_attr, face_index, face_attr, g, node_batch,
           We1, be1, lne_g, lne_b, We2, be2,
           Wn1, bn1, lnn_g, lnn_b, Wn2, bn2,
           Wg1, bg1, lng_g, lng_b, Wg2, bg2,
           norm_g, norm_b):
    row = edge_index[0]
    col = edge_index[1]
    row3 = row.reshape(_E // _BE, 1, _BE)

    nb_pad = jnp.pad(node_batch, (0, _NPAD - _N), constant_values=_B)
    nb2 = nb_pad.reshape(_NPAD // 128, 128)
    nb3 = nb_pad.reshape(_NPAD // _BN, 1, _BN)
    x_pad = jnp.pad(x, ((0, _NPAD - _N), (0, 0)))

    zeros_big = jnp.zeros((_NPAD, _DN), F32)

    Wr, Wc, Wa, W1ge = (We1[:_DN], We1[_DN:2 * _DN],
                        We1[2 * _DN:2 * _DN + _DE], We1[2 * _DN + _DE:])
    o = 0
    Wx = Wn1[o:o + _DN]; o += _DN
    Wie = Wn1[o:o + _DE]; o += _DE
    Wix = Wn1[o:o + _DN]; o += _DN
    Woe = Wn1[o:o + _DE]; o += _DE
    Wox = Wn1[o:o + _DN]; o += _DN
    W1gn = Wn1[o:]
    Wgg, Wgx, Wge = Wg1[:_DG], Wg1[_DG:_DG + _DN], Wg1[_DG + _DN:]

    r1 = lambda v: v.reshape(1, -1)

    sh, gwe, gwn = _tc_pre(nb2, g, W1ge, W1gn)

    xr, acc_in_x = _sc_gather_scatter(x, row, col, zeros_big)
    xc, acc_out_x = _sc_gather_scatter(x, col, row, zeros_big)

    e_bar, edge_new = _tc_edge(xr, xc, edge_attr, row3, sh, gwe,
                               Wr, Wc, Wa, r1(be1), r1(lne_g), r1(lne_b),
                               We2, r1(be2))

    acc_e, = _sc_scatter_e(e_bar, row, col, zeros_big)

    x_new_pad, sum_x, sum_e = _tc_node(
        x_pad, acc_in_x, acc_out_x, acc_e, nb3, gwn,
        Wx, Wie, Wix, Woe, Wox, r1(bn1), r1(lnn_g), r1(lnn_b),
        Wn2, r1(bn2), r1(norm_g), r1(norm_b))

    g_new = _tc_global(g, sum_x, sum_e, Wgg, Wgx, Wge,
                       r1(bg1), r1(lng_g), r1(lng_b), Wg2, r1(bg2))

    return (x_new_pad[:_N], edge_new, face_attr, g_new)
